# Initial kernel scaffold; baseline (speedup 1.0000x reference)
import jax
import jax.numpy as jnp
from jax.experimental import pallas as pl


def kernel(img, conv0, norm0_weight, norm0_bias, norm0_mean, norm0_var, b0_l0_norm1_weight, b0_l0_norm1_bias, b0_l0_norm1_mean, b0_l0_norm1_var, b0_l0_conv1, b0_l0_norm2_weight, b0_l0_norm2_bias, b0_l0_norm2_mean, b0_l0_norm2_var, b0_l0_conv2, b0_l1_norm1_weight, b0_l1_norm1_bias, b0_l1_norm1_mean, b0_l1_norm1_var, b0_l1_conv1, b0_l1_norm2_weight, b0_l1_norm2_bias, b0_l1_norm2_mean, b0_l1_norm2_var, b0_l1_conv2, b1_l0_norm1_weight, b1_l0_norm1_bias, b1_l0_norm1_mean, b1_l0_norm1_var, b1_l0_conv1, b1_l0_norm2_weight, b1_l0_norm2_bias, b1_l0_norm2_mean, b1_l0_norm2_var, b1_l0_conv2, b1_l1_norm1_weight, b1_l1_norm1_bias, b1_l1_norm1_mean, b1_l1_norm1_var, b1_l1_conv1, b1_l1_norm2_weight, b1_l1_norm2_bias, b1_l1_norm2_mean, b1_l1_norm2_var, b1_l1_conv2, b2_l0_norm1_weight, b2_l0_norm1_bias, b2_l0_norm1_mean, b2_l0_norm1_var, b2_l0_conv1, b2_l0_norm2_weight, b2_l0_norm2_bias, b2_l0_norm2_mean, b2_l0_norm2_var, b2_l0_conv2, b2_l1_norm1_weight, b2_l1_norm1_bias, b2_l1_norm1_mean, b2_l1_norm1_var, b2_l1_conv1, b2_l1_norm2_weight, b2_l1_norm2_bias, b2_l1_norm2_mean, b2_l1_norm2_var, b2_l1_conv2, b3_l0_norm1_weight, b3_l0_norm1_bias, b3_l0_norm1_mean, b3_l0_norm1_var, b3_l0_conv1, b3_l0_norm2_weight, b3_l0_norm2_bias, b3_l0_norm2_mean, b3_l0_norm2_var, b3_l0_conv2, b3_l1_norm1_weight, b3_l1_norm1_bias, b3_l1_norm1_mean, b3_l1_norm1_var, b3_l1_conv1, b3_l1_norm2_weight, b3_l1_norm2_bias, b3_l1_norm2_mean, b3_l1_norm2_var, b3_l1_conv2, t0_norm_weight, t0_norm_bias, t0_norm_mean, t0_norm_var, t0_conv, t1_norm_weight, t1_norm_bias, t1_norm_mean, t1_norm_var, t1_conv, t2_norm_weight, t2_norm_bias, t2_norm_mean, t2_norm_var, t2_conv, norm5_weight, norm5_bias, norm5_mean, norm5_var, classifier_w, classifier_b):
    raise NotImplementedError("write your pallas kernel here")


# speedup vs baseline: 10.2696x; 10.2696x over previous
import functools

import jax
import jax.numpy as jnp
from jax.experimental import pallas as pl
from jax.experimental.pallas import tpu as pltpu

_EPS = 1e-5
_N = 64


def _affine(w, b, m, v):
    s = w / jnp.sqrt(v + _EPS)
    t = b - m * s
    return (s.astype(jnp.float32).reshape(1, -1),
            t.astype(jnp.float32).reshape(1, -1))


def _bf(x):
    return x.astype(jnp.bfloat16)


def _bspec(shape):
    return pl.BlockSpec(shape, lambda i, _n=len(shape): (0,) * _n)


def _xspec(shape):
    return pl.BlockSpec((1,) + tuple(shape[1:]),
                        lambda i, _n=len(shape) - 1: (i,) + (0,) * _n)


_SROWS = 14


def _stem_kernel(x_ref, w_ref, s_ref, t_ref, o_ref):
    k = pl.program_id(1)
    cols = []
    for a in range(4):
        for b in range(4):
            cols.append(x_ref[0, pl.ds(_SROWS * k + a, _SROWS),
                              b:b + 112, :])
    patches = jnp.concatenate(cols, axis=-1).reshape(_SROWS * 112, 192)
    acc = jnp.dot(patches, w_ref[...], preferred_element_type=jnp.float32)
    y = jnp.maximum(acc * s_ref[...] + t_ref[...], 0.0)
    o_ref[0] = y.reshape(_SROWS, 112, 16)


def _stem(img, conv0, s, t):
    x = jnp.transpose(img, (0, 2, 3, 1))
    xp = jnp.pad(x, ((0, 0), (3, 3), (3, 3), (0, 0)))
    x12 = (xp.reshape(_N, 115, 2, 115, 2, 3)
           .transpose(0, 1, 3, 2, 4, 5)
           .reshape(_N, 115, 115, 12))
    x12 = _bf(x12)
    w8 = jnp.pad(conv0, ((0, 0), (0, 0), (0, 1), (0, 1)))
    w4 = (w8.reshape(16, 3, 4, 2, 4, 2)
          .transpose(2, 4, 3, 5, 1, 0)
          .reshape(192, 16))
    w4 = _bf(w4)
    nk = 112 // _SROWS
    return pl.pallas_call(
        _stem_kernel,
        out_shape=jax.ShapeDtypeStruct((_N, 112, 112, 16), jnp.float32),
        grid=(_N, nk),
        in_specs=[pl.BlockSpec((1, 115, 115, 12), lambda i, k: (i, 0, 0, 0)),
                  pl.BlockSpec(w4.shape, lambda i, k: (0, 0)),
                  pl.BlockSpec(s.shape, lambda i, k: (0, 0)),
                  pl.BlockSpec(t.shape, lambda i, k: (0, 0))],
        out_specs=pl.BlockSpec((1, _SROWS, 112, 16),
                               lambda i, k: (i, k, 0, 0)),
        compiler_params=pltpu.CompilerParams(
            dimension_semantics=("parallel", "arbitrary")),
    )(x12, w4, s, t)


def _layer_kernel(*refs, H, W, Cin, pre, tail):
    o_ref = refs[-1]
    x_ref, s1_ref, t1_ref, w1_ref, s2_ref, t2_ref, w2_ref = refs[:7]
    extra = refs[7:-1]

    if pre == "max":
        taps = [(0, 0), (1, 0), (0, 1)]
        m = None
        for (pi, ai) in taps:
            for (qj, bj) in taps:
                g = (pi * 2 + qj) * Cin
                sl = x_ref[0, ai:ai + H, bj:bj + W, g:g + Cin]
                m = sl if m is None else jnp.maximum(m, sl)
        x = m
    elif pre == "avg":
        xx = x_ref[0]
        x = (xx[:, :, 0 * Cin:1 * Cin] + xx[:, :, 1 * Cin:2 * Cin]
             + xx[:, :, 2 * Cin:3 * Cin] + xx[:, :, 3 * Cin:4 * Cin]) * 0.25
    else:
        x = x_ref[0]

    t = jnp.maximum(x * s1_ref[...] + t1_ref[...], 0.0)
    b_ = jnp.dot(_bf(t.reshape(H * W, Cin)), w1_ref[...],
                 preferred_element_type=jnp.float32)
    b_ = jnp.maximum(b_ * s2_ref[...] + t2_ref[...], 0.0)
    b3 = _bf(b_).reshape(H, W, 32)
    zr = jnp.zeros((1, W, 32), jnp.bfloat16)
    bp = jnp.concatenate([zr, b3, zr], axis=0)
    zc = jnp.zeros((H + 2, 1, 32), jnp.bfloat16)
    bp = jnp.concatenate([zc, bp, zc], axis=1)
    acc = jnp.zeros((H * W, 8), jnp.float32)
    for ki in range(3):
        for kj in range(3):
            sl = bp[ki:ki + H, kj:kj + W, :].reshape(H * W, 32)
            acc += jnp.dot(sl, w2_ref[ki, kj],
                           preferred_element_type=jnp.float32)
    out = jnp.concatenate([x, acc.reshape(H, W, 8)], axis=-1)

    if tail == "trans":
        sT_ref, tT_ref, wT_ref = extra
        tt = jnp.maximum(out * sT_ref[...] + tT_ref[...], 0.0)
        y = jnp.dot(_bf(tt.reshape(H * W, Cin + 8)), wT_ref[...],
                    preferred_element_type=jnp.float32)
        o_ref[0] = y.reshape(H, W, 16)
    elif tail == "head":
        s5_ref, t5_ref, cw_ref, cb_ref = extra
        tt = jnp.maximum(out * s5_ref[...] + t5_ref[...], 0.0)
        feats = jnp.mean(tt.reshape(H * W, Cin + 8), axis=0, keepdims=True)
        logit = jnp.sum(feats * cw_ref[...]) + cb_ref[0, 0]
        o_ref[...] = jnp.full((1, 1, 128), logit, jnp.float32)
    else:
        o_ref[0] = out


def _layer(x, bn1, w1, bn2, w2, H, W, Cin, pre, tail, extra):
    ins = [x, bn1[0], bn1[1], _bf(jnp.transpose(w1[:, :, 0, 0])),
           bn2[0], bn2[1], _bf(jnp.transpose(w2, (2, 3, 1, 0)))] + extra
    if tail == "head":
        out_shape = jax.ShapeDtypeStruct((_N, 1, 128), jnp.float32)
        out_spec = pl.BlockSpec((1, 1, 128), lambda i: (i, 0, 0))
    elif tail == "trans":
        out_shape = jax.ShapeDtypeStruct((_N, H, W, 16), jnp.float32)
        out_spec = _xspec((_N, H, W, 16))
    else:
        out_shape = jax.ShapeDtypeStruct((_N, H, W, Cin + 8), jnp.float32)
        out_spec = _xspec((_N, H, W, Cin + 8))
    specs = [_xspec(ins[0].shape)] + [_bspec(a.shape) for a in ins[1:]]
    return pl.pallas_call(
        functools.partial(_layer_kernel, H=H, W=W, Cin=Cin,
                          pre=pre, tail=tail),
        out_shape=out_shape,
        grid=(_N,),
        in_specs=specs,
        out_specs=out_spec,
        compiler_params=pltpu.CompilerParams(
            dimension_semantics=("parallel",)),
    )(*ins)


def _parity4(x, pad=0, pad_value=0.0):
    n, h2, w2, c = x.shape
    if pad:
        x = jnp.pad(x, ((0, 0), (pad, pad), (pad, pad), (0, 0)),
                    constant_values=pad_value)
        h2, w2 = h2 + 2 * pad, w2 + 2 * pad
    return (x.reshape(n, h2 // 2, 2, w2 // 2, 2, c)
            .transpose(0, 1, 3, 2, 4, 5)
            .reshape(n, h2 // 2, w2 // 2, 4 * c))


def kernel(img, conv0, norm0_weight, norm0_bias, norm0_mean, norm0_var, b0_l0_norm1_weight, b0_l0_norm1_bias, b0_l0_norm1_mean, b0_l0_norm1_var, b0_l0_conv1, b0_l0_norm2_weight, b0_l0_norm2_bias, b0_l0_norm2_mean, b0_l0_norm2_var, b0_l0_conv2, b0_l1_norm1_weight, b0_l1_norm1_bias, b0_l1_norm1_mean, b0_l1_norm1_var, b0_l1_conv1, b0_l1_norm2_weight, b0_l1_norm2_bias, b0_l1_norm2_mean, b0_l1_norm2_var, b0_l1_conv2, b1_l0_norm1_weight, b1_l0_norm1_bias, b1_l0_norm1_mean, b1_l0_norm1_var, b1_l0_conv1, b1_l0_norm2_weight, b1_l0_norm2_bias, b1_l0_norm2_mean, b1_l0_norm2_var, b1_l0_conv2, b1_l1_norm1_weight, b1_l1_norm1_bias, b1_l1_norm1_mean, b1_l1_norm1_var, b1_l1_conv1, b1_l1_norm2_weight, b1_l1_norm2_bias, b1_l1_norm2_mean, b1_l1_norm2_var, b1_l1_conv2, b2_l0_norm1_weight, b2_l0_norm1_bias, b2_l0_norm1_mean, b2_l0_norm1_var, b2_l0_conv1, b2_l0_norm2_weight, b2_l0_norm2_bias, b2_l0_norm2_mean, b2_l0_norm2_var, b2_l0_conv2, b2_l1_norm1_weight, b2_l1_norm1_bias, b2_l1_norm1_mean, b2_l1_norm1_var, b2_l1_conv1, b2_l1_norm2_weight, b2_l1_norm2_bias, b2_l1_norm2_mean, b2_l1_norm2_var, b2_l1_conv2, b3_l0_norm1_weight, b3_l0_norm1_bias, b3_l0_norm1_mean, b3_l0_norm1_var, b3_l0_conv1, b3_l0_norm2_weight, b3_l0_norm2_bias, b3_l0_norm2_mean, b3_l0_norm2_var, b3_l0_conv2, b3_l1_norm1_weight, b3_l1_norm1_bias, b3_l1_norm1_mean, b3_l1_norm1_var, b3_l1_conv1, b3_l1_norm2_weight, b3_l1_norm2_bias, b3_l1_norm2_mean, b3_l1_norm2_var, b3_l1_conv2, t0_norm_weight, t0_norm_bias, t0_norm_mean, t0_norm_var, t0_conv, t1_norm_weight, t1_norm_bias, t1_norm_mean, t1_norm_var, t1_conv, t2_norm_weight, t2_norm_bias, t2_norm_mean, t2_norm_var, t2_conv, norm5_weight, norm5_bias, norm5_mean, norm5_var, classifier_w, classifier_b):
    s0, t0 = _affine(norm0_weight, norm0_bias, norm0_mean, norm0_var)
    y = _stem(img, conv0, s0, t0)
    x = _parity4(y, pad=1, pad_value=-1e30)

    layers = [
        (b0_l0_norm1_weight, b0_l0_norm1_bias, b0_l0_norm1_mean, b0_l0_norm1_var,
         b0_l0_conv1, b0_l0_norm2_weight, b0_l0_norm2_bias, b0_l0_norm2_mean,
         b0_l0_norm2_var, b0_l0_conv2),
        (b0_l1_norm1_weight, b0_l1_norm1_bias, b0_l1_norm1_mean, b0_l1_norm1_var,
         b0_l1_conv1, b0_l1_norm2_weight, b0_l1_norm2_bias, b0_l1_norm2_mean,
         b0_l1_norm2_var, b0_l1_conv2),
        (b1_l0_norm1_weight, b1_l0_norm1_bias, b1_l0_norm1_mean, b1_l0_norm1_var,
         b1_l0_conv1, b1_l0_norm2_weight, b1_l0_norm2_bias, b1_l0_norm2_mean,
         b1_l0_norm2_var, b1_l0_conv2),
        (b1_l1_norm1_weight, b1_l1_norm1_bias, b1_l1_norm1_mean, b1_l1_norm1_var,
         b1_l1_conv1, b1_l1_norm2_weight, b1_l1_norm2_bias, b1_l1_norm2_mean,
         b1_l1_norm2_var, b1_l1_conv2),
        (b2_l0_norm1_weight, b2_l0_norm1_bias, b2_l0_norm1_mean, b2_l0_norm1_var,
         b2_l0_conv1, b2_l0_norm2_weight, b2_l0_norm2_bias, b2_l0_norm2_mean,
         b2_l0_norm2_var, b2_l0_conv2),
        (b2_l1_norm1_weight, b2_l1_norm1_bias, b2_l1_norm1_mean, b2_l1_norm1_var,
         b2_l1_conv1, b2_l1_norm2_weight, b2_l1_norm2_bias, b2_l1_norm2_mean,
         b2_l1_norm2_var, b2_l1_conv2),
        (b3_l0_norm1_weight, b3_l0_norm1_bias, b3_l0_norm1_mean, b3_l0_norm1_var,
         b3_l0_conv1, b3_l0_norm2_weight, b3_l0_norm2_bias, b3_l0_norm2_mean,
         b3_l0_norm2_var, b3_l0_conv2),
        (b3_l1_norm1_weight, b3_l1_norm1_bias, b3_l1_norm1_mean, b3_l1_norm1_var,
         b3_l1_conv1, b3_l1_norm2_weight, b3_l1_norm2_bias, b3_l1_norm2_mean,
         b3_l1_norm2_var, b3_l1_conv2),
    ]
    trans = [
        (t0_norm_weight, t0_norm_bias, t0_norm_mean, t0_norm_var, t0_conv),
        (t1_norm_weight, t1_norm_bias, t1_norm_mean, t1_norm_var, t1_conv),
        (t2_norm_weight, t2_norm_bias, t2_norm_mean, t2_norm_var, t2_conv),
    ]
    hw = [56, 28, 14, 7]
    for bi in range(4):
        H = hw[bi]
        l0, l1 = layers[2 * bi], layers[2 * bi + 1]
        pre0 = "max" if bi == 0 else "avg"
        bn1 = _affine(*l0[0:4])
        bn2 = _affine(*l0[5:9])
        x = _layer(x, bn1, l0[4], bn2, l0[9], H, H, 16, pre0, "plain", [])
        bn1 = _affine(*l1[0:4])
        bn2 = _affine(*l1[5:9])
        if bi < 3:
            sT, tT = _affine(*trans[bi][0:4])
            wT = _bf(jnp.transpose(trans[bi][4][:, :, 0, 0]))
            x = _layer(x, bn1, l1[4], bn2, l1[9], H, H, 24, "plain", "trans",
                       [sT, tT, wT])
            x = _parity4(x)
        else:
            s5, t5 = _affine(norm5_weight, norm5_bias, norm5_mean, norm5_var)
            cw = classifier_w.astype(jnp.float32)
            cb = classifier_b.astype(jnp.float32).reshape(1, 1)
            x = _layer(x, bn1, l1[4], bn2, l1[9], H, H, 24, "plain", "head",
                       [s5, t5, cw, cb])
    return x[:, 0, :1]
